# baseline (speedup 1.0000x reference)
import functools

import jax
import jax.numpy as jnp
from jax import lax
from jax.experimental import pallas as pl
from jax.experimental.pallas import tpu as pltpu
from jax.experimental.pallas import tpu_sc as plsc

NC, NS = 2, 16
NW = NC * NS

STEPS = 3
EIN = 4
EH = 32


def _hp(h):
    return ((h + 15) // 16) * 16


def _edge_split(n_rows):
    rpw = -(-n_rows // NW)
    rpw = ((rpw + 7) // 8) * 8
    while n_rows % rpw:
        rpw += 8
    nw_used = n_rows // rpw
    chunk = None
    for c in range(min(rpw, 128), 7, -8):
        if rpw % c == 0:
            chunk = c
            break
    return rpw, nw_used, chunk, rpw // chunk


def _sc_mesh():
    return plsc.VectorSubcoreMesh(
        core_axis_name="c", subcore_axis_name="s", num_cores=NC, num_subcores=NS
    )


_SC_PARAMS = pltpu.CompilerParams(use_tc_tiling_on_sc=False)


@functools.cache
def _gather_fn(n_rows, n_table, feat):
    rpw, nw_used, chunk, nchunks = _edge_split(n_rows)

    @functools.partial(
        pl.kernel,
        mesh=_sc_mesh(),
        out_type=jax.ShapeDtypeStruct((n_rows, feat), jnp.float32),
        scratch_types=[
            pltpu.VMEM((chunk,), jnp.int32),
            pltpu.VMEM((chunk, feat), jnp.float32),
            pltpu.SemaphoreType.DMA,
        ],
        compiler_params=_SC_PARAMS,
        name=f"sc_gather_{n_rows}x{feat}",
    )
    def gk(table_hbm, idx_hbm, out_hbm, idx_v, rows_v, sem):
        wid = lax.axis_index("s") * NC + lax.axis_index("c")

        def work():
            base = wid * rpw

            def body(j, carry):
                off = base + j * chunk
                pltpu.sync_copy(idx_hbm.at[pl.ds(off, chunk)], idx_v)
                pltpu.async_copy(table_hbm.at[idx_v], rows_v, sem).wait()
                pltpu.sync_copy(rows_v, out_hbm.at[pl.ds(off, chunk)])
                return carry

            lax.fori_loop(0, nchunks, body, 0)

        if nw_used == NW:
            work()
        else:
            pl.when(wid < nw_used)(work)

    return gk


@functools.cache
def _scatter_fn(n_rows, n_out, feat):
    rpw, nw_used, chunk, nchunks = _edge_split(n_rows)
    assert n_out % NS == 0
    rz = n_out // NS

    @functools.partial(
        pl.kernel,
        mesh=_sc_mesh(),
        out_type=jax.ShapeDtypeStruct((NC, n_out, feat), jnp.float32),
        scratch_types=[
            pltpu.VMEM((chunk,), jnp.int32),
            pltpu.VMEM((chunk, feat), jnp.float32),
            pltpu.VMEM_SHARED((n_out, feat), jnp.float32),
        ],
        compiler_params=_SC_PARAMS,
        name=f"sc_scatter_{n_rows}x{feat}_to_{n_out}",
    )
    def sk(msg_hbm, idx_hbm, zero_hbm, out_hbm, idx_v, rows_v, acc_sh):
        cid = lax.axis_index("c")
        sid = lax.axis_index("s")
        wid = sid * NC + cid
        pltpu.sync_copy(
            zero_hbm.at[pl.ds(sid * rz, rz)], acc_sh.at[pl.ds(sid * rz, rz)]
        )
        plsc.subcore_barrier()

        def work():
            base = wid * rpw

            def body(j, carry):
                off = base + j * chunk
                pltpu.sync_copy(idx_hbm.at[pl.ds(off, chunk)], idx_v)
                pltpu.sync_copy(msg_hbm.at[pl.ds(off, chunk)], rows_v)
                pltpu.sync_copy(rows_v, acc_sh.at[idx_v], add=True)
                return carry

            lax.fori_loop(0, nchunks, body, 0)

        if nw_used == NW:
            work()
        else:
            pl.when(wid < nw_used)(work)
        plsc.subcore_barrier()
        pltpu.sync_copy(
            acc_sh.at[pl.ds(sid * rz, rz)], out_hbm.at[cid, pl.ds(sid * rz, rz)]
        )

    return sk


def _sc_gather(table, idx):
    n_table, feat = table.shape
    return _gather_fn(idx.shape[0], n_table, feat)(table, idx)


def _sc_scatter(rows, idx, n_out):
    n_rows, feat = rows.shape
    zeros = jnp.zeros((n_out, feat), jnp.float32)
    return _scatter_fn(n_rows, n_out, feat)(rows, idx, zeros)


def _pick_block(m, limit):
    if m <= limit:
        return m
    for d in range((limit // 8) * 8, 7, -8):
        if m % d == 0:
            return d
    return m


@functools.cache
def _linear_fn(m, k, n, mb, relu):
    def body(x_ref, w_ref, b_ref, o_ref):
        y = jnp.dot(x_ref[...], w_ref[...], preferred_element_type=jnp.float32)
        y = y + b_ref[...]
        if relu:
            y = jnp.maximum(y, 0.0)
        o_ref[...] = y

    return pl.pallas_call(
        body,
        grid=(m // mb,),
        in_specs=[
            pl.BlockSpec((mb, k), lambda i: (i, 0)),
            pl.BlockSpec((k, n), lambda i: (0, 0)),
            pl.BlockSpec((1, n), lambda i: (0, 0)),
        ],
        out_specs=pl.BlockSpec((mb, n), lambda i: (i, 0)),
        out_shape=jax.ShapeDtypeStruct((m, n), jnp.float32),
        name=f"linear_{m}x{k}x{n}",
    )


def _linear(x, w, b, relu=False):
    m, k = x.shape
    n = w.shape[1]
    mb = _pick_block(m, 8192)
    return _linear_fn(m, k, n, mb, relu)(x, w, b.reshape(1, n))


@functools.cache
def _mlp2_fn(m, k, h, n, mb):
    def body(x_ref, w1_ref, b1_ref, w2_ref, b2_ref, o_ref):
        t = jnp.dot(x_ref[...], w1_ref[...], preferred_element_type=jnp.float32)
        t = jnp.maximum(t + b1_ref[...], 0.0)
        y = jnp.dot(t, w2_ref[...], preferred_element_type=jnp.float32)
        o_ref[...] = y + b2_ref[...]

    return pl.pallas_call(
        body,
        grid=(m // mb,),
        in_specs=[
            pl.BlockSpec((mb, k), lambda i: (i, 0)),
            pl.BlockSpec((k, h), lambda i: (0, 0)),
            pl.BlockSpec((1, h), lambda i: (0, 0)),
            pl.BlockSpec((h, n), lambda i: (0, 0)),
            pl.BlockSpec((1, n), lambda i: (0, 0)),
        ],
        out_specs=pl.BlockSpec((mb, n), lambda i: (i, 0)),
        out_shape=jax.ShapeDtypeStruct((m, n), jnp.float32),
        name=f"mlp2_{m}x{k}x{h}x{n}",
    )


def _mlp2(x, w1, b1, w2, b2):
    m, k = x.shape
    h = w1.shape[1]
    n = w2.shape[1]
    mb = _pick_block(m, 8192)
    return _mlp2_fn(m, k, h, n, mb)(
        x, w1, b1.reshape(1, h), w2, b2.reshape(1, n)
    )


@functools.cache
def _msg_fn(e, hid, hpad, eb):
    cols = hid * hpad

    def body(ef_ref, ns_ref, w1_ref, b1_ref, w2_ref, b2_ref, o_ref):
        eh = jnp.dot(ef_ref[...], w1_ref[...], preferred_element_type=jnp.float32)
        eh = jnp.maximum(eh + b1_ref[...], 0.0)
        ew = jnp.dot(eh, w2_ref[...], preferred_element_type=jnp.float32)
        ew = ew + b2_ref[...]
        ns = ns_ref[...]
        acc = ns[:, 0:1] * ew[:, 0:hpad]
        for i in range(1, hid):
            acc = acc + ns[:, i : i + 1] * ew[:, i * hpad : (i + 1) * hpad]
        o_ref[...] = acc

    return pl.pallas_call(
        body,
        grid=(e // eb,),
        in_specs=[
            pl.BlockSpec((eb, EIN), lambda i: (i, 0)),
            pl.BlockSpec((eb, hpad), lambda i: (i, 0)),
            pl.BlockSpec((EIN, EH), lambda i: (0, 0)),
            pl.BlockSpec((1, EH), lambda i: (0, 0)),
            pl.BlockSpec((EH, cols), lambda i: (0, 0)),
            pl.BlockSpec((1, cols), lambda i: (0, 0)),
        ],
        out_specs=pl.BlockSpec((eb, hpad), lambda i: (i, 0)),
        out_shape=jax.ShapeDtypeStruct((e, hpad), jnp.float32),
        name=f"msg_{e}x{hid}",
    )


def _pick_eb(e, cols):
    budget = (5 * 1024 * 1024) // (4 * cols)
    return _pick_block(e, max(8, min(e, budget)))


@functools.cache
def _gru_fn(n, hid, hpad, nb, dec):
    def body(*refs):
        if dec:
            (part_ref, deg_ref, h_ref, cb_ref, wih_ref, bih_ref, whh_ref,
             bhh_ref, d1w_ref, d1b_ref, d2w_ref, d2b_ref, o_ref) = refs
        else:
            (part_ref, deg_ref, h_ref, cb_ref, wih_ref, bih_ref, whh_ref,
             bhh_ref, o_ref) = refs
        parts = part_ref[...]
        agg = parts[0] + parts[1]
        dparts = deg_ref[...]
        d = dparts[0][:, 0:1] + dparts[1][:, 0:1]
        deg = jnp.maximum(d, 1.0)
        m = jnp.maximum(agg / deg + cb_ref[...], 0.0)
        h = h_ref[...]
        gi = jnp.dot(m, wih_ref[...], preferred_element_type=jnp.float32)
        gi = gi + bih_ref[...]
        gh = jnp.dot(h, whh_ref[...], preferred_element_type=jnp.float32)
        gh = gh + bhh_ref[...]
        ir, iz, inn = gi[:, :hpad], gi[:, hpad : 2 * hpad], gi[:, 2 * hpad :]
        hr, hz, hn = gh[:, :hpad], gh[:, hpad : 2 * hpad], gh[:, 2 * hpad :]
        r = 1.0 / (1.0 + jnp.exp(-(ir + hr)))
        z = 1.0 / (1.0 + jnp.exp(-(iz + hz)))
        nn = jnp.tanh(inn + r * hn)
        newh = (1.0 - z) * nn + z * h
        if dec:
            t = jnp.dot(newh, d1w_ref[...], preferred_element_type=jnp.float32)
            t = jnp.maximum(t + d1b_ref[...], 0.0)
            y = jnp.dot(t, d2w_ref[...], preferred_element_type=jnp.float32)
            o_ref[...] = y + d2b_ref[...]
        else:
            o_ref[...] = newh

    n_out = 3 if dec else hpad
    in_specs = [
        pl.BlockSpec((NC, nb, hpad), lambda i: (0, i, 0)),
        pl.BlockSpec((NC, nb, 16), lambda i: (0, i, 0)),
        pl.BlockSpec((nb, hpad), lambda i: (i, 0)),
        pl.BlockSpec((1, hpad), lambda i: (0, 0)),
        pl.BlockSpec((hpad, 3 * hpad), lambda i: (0, 0)),
        pl.BlockSpec((1, 3 * hpad), lambda i: (0, 0)),
        pl.BlockSpec((hpad, 3 * hpad), lambda i: (0, 0)),
        pl.BlockSpec((1, 3 * hpad), lambda i: (0, 0)),
    ]
    if dec:
        in_specs += [
            pl.BlockSpec((hpad, hid), lambda i: (0, 0)),
            pl.BlockSpec((1, hid), lambda i: (0, 0)),
            pl.BlockSpec((hid, 3), lambda i: (0, 0)),
            pl.BlockSpec((1, 3), lambda i: (0, 0)),
        ]
    return pl.pallas_call(
        body,
        grid=(n // nb,),
        in_specs=in_specs,
        out_specs=pl.BlockSpec((nb, n_out), lambda i: (i, 0)),
        out_shape=jax.ShapeDtypeStruct((n, n_out), jnp.float32),
        name=f"gru_{n}x{hid}{'_dec' if dec else ''}",
    )


def _pad_cols(w, to):
    return jnp.pad(w, ((0, 0), (0, to - w.shape[1])))


def _mpnn_level(p, x, ef, src, dst, n, hid, deg_parts, dec=False):
    e = ef.shape[0]
    hpad = _hp(hid)

    pn1T = p["pn1_w"].T
    pn2Tp = _pad_cols(p["pn2_w"].T, hpad)
    pn2bp = jnp.pad(p["pn2_b"], (0, hpad - hid))
    en1T = p["en1_w"].T
    en2Tp = jnp.pad(
        p["en2_w"].T.reshape(EH, hid, hid), ((0, 0), (0, 0), (0, hpad - hid))
    ).reshape(EH, hid * hpad)
    en2bp = jnp.pad(
        p["en2_b"].reshape(hid, hid), ((0, 0), (0, hpad - hid))
    ).reshape(hid * hpad)
    cbp = jnp.pad(p["conv_b"], (0, hpad - hid)).reshape(1, hpad)
    wihTp = jnp.pad(
        p["gru_wih"].reshape(3, hid, hid).transpose(2, 0, 1),
        ((0, hpad - hid), (0, 0), (0, hpad - hid)),
    ).reshape(hpad, 3 * hpad)
    bihp = jnp.pad(
        p["gru_bih"].reshape(3, hid), ((0, 0), (0, hpad - hid))
    ).reshape(1, 3 * hpad)
    whhTp = jnp.pad(
        p["gru_whh"].reshape(3, hid, hid).transpose(2, 0, 1),
        ((0, hpad - hid), (0, 0), (0, hpad - hid)),
    ).reshape(hpad, 3 * hpad)
    bhhp = jnp.pad(
        p["gru_bhh"].reshape(3, hid), ((0, 0), (0, hpad - hid))
    ).reshape(1, 3 * hpad)

    node = _mlp2(x, pn1T, p["pn1_b"], pn2Tp, pn2bp)

    eb = _pick_eb(e, hid * hpad)
    msg_call = _msg_fn(e, hid, hpad, eb)
    nb = _pick_block(n, 4096)
    gru_extra = (
        (p["dec1_w"].T, p["dec1_b"].reshape(1, hid), p["dec2_w"].T,
         p["dec2_b"].reshape(1, 3))
        if dec
        else ()
    )

    out = node
    for step in range(STEPS):
        nsrc = _sc_gather(out, src)
        msg = msg_call(
            ef, nsrc, en1T, p["en1_b"].reshape(1, EH), en2Tp,
            en2bp.reshape(1, hid * hpad)
        )
        parts = _sc_scatter(msg, dst, n)
        last = step == STEPS - 1
        if dec and last:
            out = _gru_fn(n, hid, hpad, nb, True)(
                parts, deg_parts, out, cbp, wihTp, bihp, whhTp, bhhp, *gru_extra
            )
        else:
            out = _gru_fn(n, hid, hpad, nb, False)(
                parts, deg_parts, out, cbp, wihTp, bihp, whhTp, bhhp
            )
    return out


def _pool_mats(g, f):
    a = jnp.repeat(jnp.eye(g // f, dtype=jnp.float32), f, axis=0) / f
    b = jnp.tile(jnp.eye(g // f, dtype=jnp.float32), (f, 1)) / f
    return a, b


def _pool(x_nodes, c, g, f):
    go = g // f
    cm = x_nodes.reshape(6, g, g, c).transpose(3, 0, 1, 2).reshape(c * 6 * g, g)
    a, b = _pool_mats(g, f)
    zb = jnp.zeros((go,), jnp.float32)
    t = _linear(cm, a, zb)
    t = t.reshape(c * 6 * go, f * go)
    t = _linear(t, b, zb)
    return t.reshape(c, 6 * go * go).T


def _convT(x_nodes, w, b, g):
    cin, d = w.shape[0], w.shape[1]
    wc = jnp.transpose(w, (0, 2, 3, 1)).reshape(cin, 4 * d)
    bc = jnp.tile(b, 4)
    z = _linear(x_nodes, wc, bc)
    z = z.reshape(6, g, g, 2, 2, d).transpose(0, 1, 3, 2, 4, 5)
    return z.reshape(6 * 2 * g * 2 * g, d)


def kernel(in_feat, edge1, edge2, edge3, edge4, edge5, params, edge_index1,
           edge_index3, edge_index4, edge_index5):
    del edge2
    n1 = in_feat.shape[0]
    n3 = n1 // 16
    n4 = n3 // 4
    n5 = n4 // 4
    p = params

    src1, dst1 = edge_index1[0], edge_index1[1]
    src3, dst3 = edge_index3[0], edge_index3[1]
    src4, dst4 = edge_index4[0], edge_index4[1]
    src5, dst5 = edge_index5[0], edge_index5[1]

    deg1 = _sc_scatter(jnp.ones((edge1.shape[0], 16), jnp.float32), dst1, n1)
    deg3 = _sc_scatter(jnp.ones((edge3.shape[0], 16), jnp.float32), dst3, n3)
    deg4 = _sc_scatter(jnp.ones((edge4.shape[0], 16), jnp.float32), dst4, n4)
    deg5 = _sc_scatter(jnp.ones((edge5.shape[0], 16), jnp.float32), dst5, n5)

    h2in = _pool(in_feat, 7, 64, 4)
    h2o = _mpnn_level(p["mp1"], h2in, edge3, src3, dst3, n3, 32, deg3)
    h3in = _pool(h2o, 32, 16, 2)
    h3o = _mpnn_level(p["mp2"], h3in, edge4, src4, dst4, n4, 64, deg4)
    h4in = _pool(h3o, 64, 8, 2)
    h4o = _mpnn_level(p["mp3"], h4in, edge5, src5, dst5, n5, 128, deg5)

    u1 = _convT(h4o, p["up1_w"], p["up1_b"], 4)
    h6in = jnp.concatenate([u1, h3o], axis=1)
    h6o = _mpnn_level(p["mp4"], h6in, edge4, src4, dst4, n4, 98, deg4)
    u2 = _convT(h6o[:, :98], p["up2_w"], p["up2_b"], 8)
    h7in = jnp.concatenate([u2, h2o], axis=1)
    h7o = _mpnn_level(p["mp5"], h7in, edge3, src3, dst3, n3, 60, deg3)
    u3 = _convT(h7o[:, :60], p["up3_w"], p["up3_b"], 16)
    u4 = _convT(u3, p["up4_w"], p["up4_b"], 32)
    h8in = jnp.concatenate([u4, in_feat], axis=1)
    return _mpnn_level(p["mp6"], h8in, edge1, src1, dst1, n1, 32, deg1, dec=True)


# speedup vs baseline: 2.7888x; 2.7888x over previous
import functools

import jax
import jax.numpy as jnp
from jax import lax
from jax.experimental import pallas as pl
from jax.experimental.pallas import tpu as pltpu
from jax.experimental.pallas import tpu_sc as plsc

NC, NS = 2, 16
NW = NC * NS

STEPS = 3
EIN = 4
EH = 32


def _hp(h):
    return ((h + 15) // 16) * 16


def _edge_split(n_rows):
    rpw = -(-n_rows // NW)
    rpw = ((rpw + 7) // 8) * 8
    while n_rows % rpw:
        rpw += 8
    nw_used = n_rows // rpw
    chunk = None
    for c in range(min(rpw, 128), 7, -8):
        if rpw % c == 0:
            chunk = c
            break
    return rpw, nw_used, chunk, rpw // chunk


def _sc_mesh():
    return plsc.VectorSubcoreMesh(
        core_axis_name="c", subcore_axis_name="s", num_cores=NC, num_subcores=NS
    )


_SC_PARAMS = pltpu.CompilerParams(use_tc_tiling_on_sc=False)


@functools.cache
def _gather_fn(n_rows, n_table, feat):
    rpw, nw_used, chunk, nchunks = _edge_split(n_rows)

    @functools.partial(
        pl.kernel,
        mesh=_sc_mesh(),
        out_type=jax.ShapeDtypeStruct((n_rows, feat), jnp.float32),
        scratch_types=[
            pltpu.VMEM((chunk,), jnp.int32),
            pltpu.VMEM((chunk, feat), jnp.float32),
            pltpu.SemaphoreType.DMA,
        ],
        compiler_params=_SC_PARAMS,
        name=f"sc_gather_{n_rows}x{feat}",
    )
    def gk(table_hbm, idx_hbm, out_hbm, idx_v, rows_v, sem):
        wid = lax.axis_index("s") * NC + lax.axis_index("c")

        def work():
            base = wid * rpw

            def body(j, carry):
                off = base + j * chunk
                pltpu.sync_copy(idx_hbm.at[pl.ds(off, chunk)], idx_v)
                pltpu.async_copy(table_hbm.at[idx_v], rows_v, sem).wait()
                pltpu.sync_copy(rows_v, out_hbm.at[pl.ds(off, chunk)])
                return carry

            lax.fori_loop(0, nchunks, body, 0)

        if nw_used == NW:
            work()
        else:
            pl.when(wid < nw_used)(work)

    return gk


@functools.cache
def _scatter_fn(n_rows, n_out, feat):
    rpw, nw_used, chunk, nchunks = _edge_split(n_rows)
    assert n_out % NS == 0
    rz = n_out // NS

    @functools.partial(
        pl.kernel,
        mesh=_sc_mesh(),
        out_type=jax.ShapeDtypeStruct((NC, n_out, feat), jnp.float32),
        scratch_types=[
            pltpu.VMEM((chunk,), jnp.int32),
            pltpu.VMEM((chunk, feat), jnp.float32),
            pltpu.VMEM_SHARED((n_out, feat), jnp.float32),
        ],
        compiler_params=_SC_PARAMS,
        name=f"sc_scatter_{n_rows}x{feat}_to_{n_out}",
    )
    def sk(msg_hbm, idx_hbm, zero_hbm, out_hbm, idx_v, rows_v, acc_sh):
        cid = lax.axis_index("c")
        sid = lax.axis_index("s")
        wid = sid * NC + cid
        pltpu.sync_copy(
            zero_hbm.at[pl.ds(sid * rz, rz)], acc_sh.at[pl.ds(sid * rz, rz)]
        )
        plsc.subcore_barrier()

        def work():
            base = wid * rpw

            def body(j, carry):
                off = base + j * chunk
                pltpu.sync_copy(idx_hbm.at[pl.ds(off, chunk)], idx_v)
                pltpu.sync_copy(msg_hbm.at[pl.ds(off, chunk)], rows_v)
                pltpu.sync_copy(rows_v, acc_sh.at[idx_v], add=True)
                return carry

            lax.fori_loop(0, nchunks, body, 0)

        if nw_used == NW:
            work()
        else:
            pl.when(wid < nw_used)(work)
        plsc.subcore_barrier()
        pltpu.sync_copy(
            acc_sh.at[pl.ds(sid * rz, rz)], out_hbm.at[cid, pl.ds(sid * rz, rz)]
        )

    return sk


def _sc_gather(table, idx):
    n_table, feat = table.shape
    return _gather_fn(idx.shape[0], n_table, feat)(table, idx)


def _sc_scatter(rows, idx, n_out):
    n_rows, feat = rows.shape
    zeros = jnp.zeros((n_out, feat), jnp.float32)
    return _scatter_fn(n_rows, n_out, feat)(rows, idx, zeros)


def _pick_block(m, limit):
    if m <= limit:
        return m
    for d in range((limit // 8) * 8, 7, -8):
        if m % d == 0:
            return d
    return m


@functools.cache
def _linear_fn(m, k, n, mb, relu):
    def body(x_ref, w_ref, b_ref, o_ref):
        y = jnp.dot(x_ref[...], w_ref[...], preferred_element_type=jnp.float32)
        y = y + b_ref[...]
        if relu:
            y = jnp.maximum(y, 0.0)
        o_ref[...] = y

    return pl.pallas_call(
        body,
        grid=(m // mb,),
        in_specs=[
            pl.BlockSpec((mb, k), lambda i: (i, 0)),
            pl.BlockSpec((k, n), lambda i: (0, 0)),
            pl.BlockSpec((1, n), lambda i: (0, 0)),
        ],
        out_specs=pl.BlockSpec((mb, n), lambda i: (i, 0)),
        out_shape=jax.ShapeDtypeStruct((m, n), jnp.float32),
        name=f"linear_{m}x{k}x{n}",
    )


def _linear(x, w, b, relu=False):
    m, k = x.shape
    n = w.shape[1]
    mb = _pick_block(m, 8192)
    return _linear_fn(m, k, n, mb, relu)(x, w, b.reshape(1, n))


@functools.cache
def _mlp2_fn(m, k, h, n, mb):
    def body(x_ref, w1_ref, b1_ref, w2_ref, b2_ref, o_ref):
        t = jnp.dot(x_ref[...], w1_ref[...], preferred_element_type=jnp.float32)
        t = jnp.maximum(t + b1_ref[...], 0.0)
        y = jnp.dot(t, w2_ref[...], preferred_element_type=jnp.float32)
        o_ref[...] = y + b2_ref[...]

    return pl.pallas_call(
        body,
        grid=(m // mb,),
        in_specs=[
            pl.BlockSpec((mb, k), lambda i: (i, 0)),
            pl.BlockSpec((k, h), lambda i: (0, 0)),
            pl.BlockSpec((1, h), lambda i: (0, 0)),
            pl.BlockSpec((h, n), lambda i: (0, 0)),
            pl.BlockSpec((1, n), lambda i: (0, 0)),
        ],
        out_specs=pl.BlockSpec((mb, n), lambda i: (i, 0)),
        out_shape=jax.ShapeDtypeStruct((m, n), jnp.float32),
        name=f"mlp2_{m}x{k}x{h}x{n}",
    )


def _mlp2(x, w1, b1, w2, b2):
    m, k = x.shape
    h = w1.shape[1]
    n = w2.shape[1]
    mb = _pick_block(m, 8192)
    return _mlp2_fn(m, k, h, n, mb)(
        x, w1, b1.reshape(1, h), w2, b2.reshape(1, n)
    )


@functools.cache
def _msg_fn(e, hid, hpad, eb):
    rows = hid * hpad

    def body(ef_ref, ns_ref, w1_ref, b1_ref, w2_ref, b2_ref, o_ref):
        eh = jnp.dot(w1_ref[...], ef_ref[...], preferred_element_type=jnp.float32)
        eh = jnp.maximum(eh + b1_ref[...], 0.0)
        ew = jnp.dot(w2_ref[...], eh, preferred_element_type=jnp.float32)
        ew = ew + b2_ref[...]
        ns = jnp.transpose(ns_ref[...])
        acc = ns[0:1, :] * ew[0:hpad, :]
        for i in range(1, hid):
            acc = acc + ns[i : i + 1, :] * ew[i * hpad : (i + 1) * hpad, :]
        o_ref[...] = jnp.transpose(acc)

    return pl.pallas_call(
        body,
        grid=(e // eb,),
        in_specs=[
            pl.BlockSpec((EIN, eb), lambda i: (0, i)),
            pl.BlockSpec((eb, hpad), lambda i: (i, 0)),
            pl.BlockSpec((EH, EIN), lambda i: (0, 0)),
            pl.BlockSpec((EH, 1), lambda i: (0, 0)),
            pl.BlockSpec((rows, EH), lambda i: (0, 0)),
            pl.BlockSpec((rows, 1), lambda i: (0, 0)),
        ],
        out_specs=pl.BlockSpec((eb, hpad), lambda i: (i, 0)),
        out_shape=jax.ShapeDtypeStruct((e, hpad), jnp.float32),
        name=f"msg_{e}x{hid}",
    )


def _pick_eb(e, cols):
    budget = (8 * 1024 * 1024) // (4 * cols)
    best = 128
    for d in range(128, min(e, budget) + 1, 128):
        if e % d == 0:
            best = d
    return best


@functools.cache
def _gru_fn(n, hid, hpad, nb, dec):
    def body(*refs):
        if dec:
            (part_ref, deg_ref, h_ref, cb_ref, wih_ref, bih_ref, whh_ref,
             bhh_ref, d1w_ref, d1b_ref, d2w_ref, d2b_ref, o_ref) = refs
        else:
            (part_ref, deg_ref, h_ref, cb_ref, wih_ref, bih_ref, whh_ref,
             bhh_ref, o_ref) = refs
        parts = part_ref[...]
        agg = parts[0] + parts[1]
        dparts = deg_ref[...]
        d = dparts[0][:, 0:1] + dparts[1][:, 0:1]
        deg = jnp.maximum(d, 1.0)
        m = jnp.maximum(agg / deg + cb_ref[...], 0.0)
        h = h_ref[...]
        gi = jnp.dot(m, wih_ref[...], preferred_element_type=jnp.float32)
        gi = gi + bih_ref[...]
        gh = jnp.dot(h, whh_ref[...], preferred_element_type=jnp.float32)
        gh = gh + bhh_ref[...]
        ir, iz, inn = gi[:, :hpad], gi[:, hpad : 2 * hpad], gi[:, 2 * hpad :]
        hr, hz, hn = gh[:, :hpad], gh[:, hpad : 2 * hpad], gh[:, 2 * hpad :]
        r = 1.0 / (1.0 + jnp.exp(-(ir + hr)))
        z = 1.0 / (1.0 + jnp.exp(-(iz + hz)))
        nn = jnp.tanh(inn + r * hn)
        newh = (1.0 - z) * nn + z * h
        if dec:
            t = jnp.dot(newh, d1w_ref[...], preferred_element_type=jnp.float32)
            t = jnp.maximum(t + d1b_ref[...], 0.0)
            y = jnp.dot(t, d2w_ref[...], preferred_element_type=jnp.float32)
            o_ref[...] = y + d2b_ref[...]
        else:
            o_ref[...] = newh

    n_out = 3 if dec else hpad
    in_specs = [
        pl.BlockSpec((NC, nb, hpad), lambda i: (0, i, 0)),
        pl.BlockSpec((NC, nb, 16), lambda i: (0, i, 0)),
        pl.BlockSpec((nb, hpad), lambda i: (i, 0)),
        pl.BlockSpec((1, hpad), lambda i: (0, 0)),
        pl.BlockSpec((hpad, 3 * hpad), lambda i: (0, 0)),
        pl.BlockSpec((1, 3 * hpad), lambda i: (0, 0)),
        pl.BlockSpec((hpad, 3 * hpad), lambda i: (0, 0)),
        pl.BlockSpec((1, 3 * hpad), lambda i: (0, 0)),
    ]
    if dec:
        in_specs += [
            pl.BlockSpec((hpad, hid), lambda i: (0, 0)),
            pl.BlockSpec((1, hid), lambda i: (0, 0)),
            pl.BlockSpec((hid, 3), lambda i: (0, 0)),
            pl.BlockSpec((1, 3), lambda i: (0, 0)),
        ]
    return pl.pallas_call(
        body,
        grid=(n // nb,),
        in_specs=in_specs,
        out_specs=pl.BlockSpec((nb, n_out), lambda i: (i, 0)),
        out_shape=jax.ShapeDtypeStruct((n, n_out), jnp.float32),
        name=f"gru_{n}x{hid}{'_dec' if dec else ''}",
    )


def _pad_cols(w, to):
    return jnp.pad(w, ((0, 0), (0, to - w.shape[1])))


def _mpnn_level(p, x, ef, src, dst, n, hid, deg_parts, dec=False):
    e = ef.shape[0]
    hpad = _hp(hid)

    pn1T = p["pn1_w"].T
    pn2Tp = _pad_cols(p["pn2_w"].T, hpad)
    pn2bp = jnp.pad(p["pn2_b"], (0, hpad - hid))
    en2p = jnp.pad(
        p["en2_w"].reshape(hid, hid, EH), ((0, 0), (0, hpad - hid), (0, 0))
    ).reshape(hid * hpad, EH)
    en2bp = jnp.pad(
        p["en2_b"].reshape(hid, hid), ((0, 0), (0, hpad - hid))
    ).reshape(hid * hpad)
    cbp = jnp.pad(p["conv_b"], (0, hpad - hid)).reshape(1, hpad)
    wihTp = jnp.pad(
        p["gru_wih"].reshape(3, hid, hid).transpose(2, 0, 1),
        ((0, hpad - hid), (0, 0), (0, hpad - hid)),
    ).reshape(hpad, 3 * hpad)
    bihp = jnp.pad(
        p["gru_bih"].reshape(3, hid), ((0, 0), (0, hpad - hid))
    ).reshape(1, 3 * hpad)
    whhTp = jnp.pad(
        p["gru_whh"].reshape(3, hid, hid).transpose(2, 0, 1),
        ((0, hpad - hid), (0, 0), (0, hpad - hid)),
    ).reshape(hpad, 3 * hpad)
    bhhp = jnp.pad(
        p["gru_bhh"].reshape(3, hid), ((0, 0), (0, hpad - hid))
    ).reshape(1, 3 * hpad)

    node = _mlp2(x, pn1T, p["pn1_b"], pn2Tp, pn2bp)

    eb = _pick_eb(e, hid * hpad)
    msg_call = _msg_fn(e, hid, hpad, eb)
    nb = _pick_block(n, 4096)
    gru_extra = (
        (p["dec1_w"].T, p["dec1_b"].reshape(1, hid), p["dec2_w"].T,
         p["dec2_b"].reshape(1, 3))
        if dec
        else ()
    )

    efT = ef.T
    out = node
    for step in range(STEPS):
        nsrc = _sc_gather(out, src)
        msg = msg_call(
            efT, nsrc, p["en1_w"], p["en1_b"].reshape(EH, 1), en2p,
            en2bp.reshape(hid * hpad, 1)
        )
        parts = _sc_scatter(msg, dst, n)
        last = step == STEPS - 1
        if dec and last:
            out = _gru_fn(n, hid, hpad, nb, True)(
                parts, deg_parts, out, cbp, wihTp, bihp, whhTp, bhhp, *gru_extra
            )
        else:
            out = _gru_fn(n, hid, hpad, nb, False)(
                parts, deg_parts, out, cbp, wihTp, bihp, whhTp, bhhp
            )
    return out


def _pool_mats(g, f):
    a = jnp.repeat(jnp.eye(g // f, dtype=jnp.float32), f, axis=0) / f
    b = jnp.tile(jnp.eye(g // f, dtype=jnp.float32), (f, 1)) / f
    return a, b


def _pool(x_nodes, c, g, f):
    go = g // f
    cm = x_nodes.reshape(6, g, g, c).transpose(3, 0, 1, 2).reshape(c * 6 * g, g)
    a, b = _pool_mats(g, f)
    zb = jnp.zeros((go,), jnp.float32)
    t = _linear(cm, a, zb)
    t = t.reshape(c * 6 * go, f * go)
    t = _linear(t, b, zb)
    return t.reshape(c, 6 * go * go).T


def _convT(x_nodes, w, b, g):
    cin, d = w.shape[0], w.shape[1]
    wc = jnp.transpose(w, (0, 2, 3, 1)).reshape(cin, 4 * d)
    bc = jnp.tile(b, 4)
    z = _linear(x_nodes, wc, bc)
    z = z.reshape(6, g, g, 2, 2, d).transpose(0, 1, 3, 2, 4, 5)
    return z.reshape(6 * 2 * g * 2 * g, d)


def kernel(in_feat, edge1, edge2, edge3, edge4, edge5, params, edge_index1,
           edge_index3, edge_index4, edge_index5):
    del edge2
    n1 = in_feat.shape[0]
    n3 = n1 // 16
    n4 = n3 // 4
    n5 = n4 // 4
    p = params

    src1, dst1 = edge_index1[0], edge_index1[1]
    src3, dst3 = edge_index3[0], edge_index3[1]
    src4, dst4 = edge_index4[0], edge_index4[1]
    src5, dst5 = edge_index5[0], edge_index5[1]

    deg1 = _sc_scatter(jnp.ones((edge1.shape[0], 16), jnp.float32), dst1, n1)
    deg3 = _sc_scatter(jnp.ones((edge3.shape[0], 16), jnp.float32), dst3, n3)
    deg4 = _sc_scatter(jnp.ones((edge4.shape[0], 16), jnp.float32), dst4, n4)
    deg5 = _sc_scatter(jnp.ones((edge5.shape[0], 16), jnp.float32), dst5, n5)

    h2in = _pool(in_feat, 7, 64, 4)
    h2o = _mpnn_level(p["mp1"], h2in, edge3, src3, dst3, n3, 32, deg3)
    h3in = _pool(h2o, 32, 16, 2)
    h3o = _mpnn_level(p["mp2"], h3in, edge4, src4, dst4, n4, 64, deg4)
    h4in = _pool(h3o, 64, 8, 2)
    h4o = _mpnn_level(p["mp3"], h4in, edge5, src5, dst5, n5, 128, deg5)

    u1 = _convT(h4o, p["up1_w"], p["up1_b"], 4)
    h6in = jnp.concatenate([u1, h3o], axis=1)
    h6o = _mpnn_level(p["mp4"], h6in, edge4, src4, dst4, n4, 98, deg4)
    u2 = _convT(h6o[:, :98], p["up2_w"], p["up2_b"], 8)
    h7in = jnp.concatenate([u2, h2o], axis=1)
    h7o = _mpnn_level(p["mp5"], h7in, edge3, src3, dst3, n3, 60, deg3)
    u3 = _convT(h7o[:, :60], p["up3_w"], p["up3_b"], 16)
    u4 = _convT(u3, p["up4_w"], p["up4_b"], 32)
    h8in = jnp.concatenate([u4, in_feat], axis=1)
    return _mpnn_level(p["mp6"], h8in, edge1, src1, dst1, n1, 32, deg1, dec=True)
